# baseline (speedup 1.0000x reference)
import functools

import jax
import jax.numpy as jnp
from jax import lax
from jax.experimental import pallas as pl
from jax.experimental.pallas import tpu as pltpu
from jax.experimental.pallas import tpu_sc as plsc

DIM = 64
SEQ = 50
JC = 2

_info = plsc.get_sparse_core_info()
_NC, _NS, _NL = _info.num_cores, _info.num_subcores, _info.num_lanes
_NW = _NC * _NS


@functools.partial(jax.jit, static_argnames=("n_seq",))
def _gather_sc(idx_flat, w, *, n_seq):
    s_per_w = n_seq // _NW
    b_per_w = s_per_w * SEQ
    nchunk = SEQ // JC
    chunk_rows = JC * s_per_w
    mesh = plsc.VectorSubcoreMesh(core_axis_name="c", subcore_axis_name="s")

    @functools.partial(
        pl.kernel,
        mesh=mesh,
        out_type=jax.ShapeDtypeStruct((SEQ, DIM, n_seq), jnp.float32),
        scratch_types=[
            pltpu.VMEM((b_per_w,), jnp.int32),
            pltpu.VMEM((b_per_w,), jnp.int32),
            pltpu.VMEM((chunk_rows, DIM), jnp.float32),
            pltpu.VMEM((chunk_rows, DIM), jnp.float32),
            pltpu.VMEM((DIM, JC * s_per_w), jnp.float32),
            pltpu.VMEM((DIM, JC * s_per_w), jnp.float32),
            pltpu.SemaphoreType.DMA,
            pltpu.SemaphoreType.DMA,
            pltpu.SemaphoreType.DMA,
            pltpu.SemaphoreType.DMA,
        ],
        compiler_params=pltpu.CompilerParams(
            use_tc_tiling_on_sc=False, needs_layout_passes=False
        ),
    )
    def k(idx_hbm, table_hbm, out_hbm, idx_v, idx_t, rows0, rows1, t0, t1,
          g0, g1, o0, o1):
        wid = lax.axis_index("s") * _NC + lax.axis_index("c")
        base = wid * b_per_w
        s0 = wid * s_per_w
        pltpu.sync_copy(idx_hbm.at[pl.ds(base, b_per_w)], idx_v)

        lanes = lax.iota(jnp.int32, _NL)
        nsb = s_per_w // _NL

        def build_idx(j, _):
            for sb in range(nsb):
                src = (sb * _NL + lanes) * SEQ + j
                v = plsc.load_gather(idx_v, [src])
                plsc.store_scatter(idx_t, [j * s_per_w + sb * _NL + lanes], v)
            return 0

        lax.fori_loop(0, SEQ, build_idx, 0)

        bufs = (rows0, rows1)
        tbufs = (t0, t1)
        gsems = (g0, g1)
        osems = (o0, o1)

        def start_gather(c, buf, sem):
            return pltpu.async_copy(
                table_hbm.at[idx_t.at[pl.ds(c * chunk_rows, chunk_rows)]],
                buf, sem,
            )

        def transpose(buf, tbuf):
            svecs = [
                jl * s_per_w + sb * _NL + lanes
                for jl in range(JC)
                for sb in range(nsb)
            ]

            def body(d, _):
                dsplat = jnp.full((_NL,), 0, jnp.int32) + d
                for sv in svecs:
                    v = plsc.load_gather(buf, [sv, dsplat])
                    plsc.store_scatter(tbuf, [dsplat, sv], v)
                return 0

            lax.fori_loop(0, DIM, body, 0)

        def start_outs(c, tbuf, sem):
            copies = []
            for jl in range(JC):
                j = c * JC + jl
                copies.append(
                    pltpu.async_copy(
                        tbuf.at[:, pl.ds(jl * s_per_w, s_per_w)],
                        out_hbm.at[j, :, pl.ds(s0, s_per_w)],
                        sem,
                    )
                )
            return copies

        gathers = [None, None]
        outs = [None, None]
        gathers[0] = start_gather(0, bufs[0], gsems[0])
        for c in range(nchunk):
            b = c & 1
            gathers[b].wait()
            if c + 1 < nchunk:
                gathers[1 - b] = start_gather(c + 1, bufs[1 - b], gsems[1 - b])
            if outs[b] is not None:
                for o in outs[b]:
                    o.wait()
            transpose(bufs[b], tbufs[b])
            outs[b] = start_outs(c, tbufs[b], osems[b])
        for grp in outs:
            if grp is not None:
                for o in grp:
                    o.wait()

    return k(idx_flat, w)


def kernel(token_ids, w):
    n_seq = token_ids.shape[0]
    idx_flat = token_ids.reshape(-1).astype(jnp.int32)
    out_t = _gather_sc(idx_flat, w, n_seq=n_seq)
    return jnp.transpose(out_t, (2, 0, 1))


# speedup vs baseline: 1.4003x; 1.4003x over previous
import functools

import jax
import jax.numpy as jnp
from jax import lax
from jax.experimental import pallas as pl
from jax.experimental.pallas import tpu as pltpu
from jax.experimental.pallas import tpu_sc as plsc

DIM = 64
SEQ = 50
JC = 2

_info = plsc.get_sparse_core_info()
_NC, _NS, _NL = _info.num_cores, _info.num_subcores, _info.num_lanes
_NW = _NC * _NS


@functools.partial(jax.jit, static_argnames=("n_seq",))
def _gather_sc(idx_flat, w, *, n_seq):
    s_per_w = n_seq // _NW
    b_per_w = s_per_w * SEQ
    nchunk = SEQ // JC
    chunk_rows = JC * s_per_w
    mesh = plsc.VectorSubcoreMesh(core_axis_name="c", subcore_axis_name="s")

    @functools.partial(
        pl.kernel,
        mesh=mesh,
        out_type=jax.ShapeDtypeStruct((SEQ, DIM, n_seq), jnp.float32),
        scratch_types=[
            pltpu.VMEM((b_per_w,), jnp.int32),
            pltpu.VMEM((b_per_w,), jnp.int32),
            pltpu.VMEM((chunk_rows, DIM), jnp.float32),
            pltpu.VMEM((chunk_rows, DIM), jnp.float32),
            pltpu.VMEM((DIM, JC * s_per_w), jnp.float32),
            pltpu.VMEM((DIM, JC * s_per_w), jnp.float32),
            pltpu.SemaphoreType.DMA,
            pltpu.SemaphoreType.DMA,
            pltpu.SemaphoreType.DMA,
            pltpu.SemaphoreType.DMA,
        ],
        compiler_params=pltpu.CompilerParams(
            use_tc_tiling_on_sc=False, needs_layout_passes=False
        ),
    )
    def k(idx_hbm, table_hbm, out_hbm, idx_v, idx_t, rows0, rows1, t0, t1,
          g0, g1, o0, o1):
        wid = lax.axis_index("s") * _NC + lax.axis_index("c")
        base = wid * b_per_w
        s0 = wid * s_per_w
        pltpu.sync_copy(idx_hbm.at[pl.ds(base, b_per_w)], idx_v)

        lanes = lax.iota(jnp.int32, _NL)
        nsb = s_per_w // _NL

        lane_seq = lanes * SEQ

        @plsc.parallel_loop(0, SEQ, unroll=2)
        def _build_idx(j):
            for sb in range(nsb):
                v = plsc.load_gather(idx_v, [lane_seq + (sb * _NL * SEQ + j)])
                plsc.store_scatter(
                    idx_t, [j * s_per_w + sb * _NL + lanes], v
                )

        bufs = (rows0, rows1)
        tbufs = (t0, t1)
        gsems = (g0, g1)
        osems = (o0, o1)

        def start_gather(c, buf, sem):
            return pltpu.async_copy(
                table_hbm.at[idx_t.at[pl.ds(c * chunk_rows, chunk_rows)]],
                buf, sem,
            )

        def transpose(buf, tbuf):
            dvecs = [d16 * _NL + lanes for d16 in range(DIM // _NL)]

            @plsc.parallel_loop(0, chunk_rows, unroll=8)
            def _tr(r):
                rsplat = jnp.full((_NL,), 0, jnp.int32) + r
                for d16, dvec in enumerate(dvecs):
                    v = buf[r, pl.ds(d16 * _NL, _NL)]
                    plsc.store_scatter(tbuf, [dvec, rsplat], v)

        def start_outs(c, tbuf, sem):
            copies = []
            for jl in range(JC):
                j = c * JC + jl
                copies.append(
                    pltpu.async_copy(
                        tbuf.at[:, pl.ds(jl * s_per_w, s_per_w)],
                        out_hbm.at[j, :, pl.ds(s0, s_per_w)],
                        sem,
                    )
                )
            return copies

        gathers = [None, None]
        outs = [None, None]
        gathers[0] = start_gather(0, bufs[0], gsems[0])
        for c in range(nchunk):
            b = c & 1
            gathers[b].wait()
            if c + 1 < nchunk:
                gathers[1 - b] = start_gather(c + 1, bufs[1 - b], gsems[1 - b])
            if outs[b] is not None:
                for o in outs[b]:
                    o.wait()
            transpose(bufs[b], tbufs[b])
            outs[b] = start_outs(c, tbufs[b], osems[b])
        for grp in outs:
            if grp is not None:
                for o in grp:
                    o.wait()

    return k(idx_flat, w)


def kernel(token_ids, w):
    n_seq = token_ids.shape[0]
    idx_flat = token_ids.reshape(-1).astype(jnp.int32)
    out_t = _gather_sc(idx_flat, w, n_seq=n_seq)
    return jnp.transpose(out_t, (2, 0, 1))


# speedup vs baseline: 2.5603x; 1.8284x over previous
import functools

import jax
import jax.numpy as jnp
from jax import lax
from jax.experimental import pallas as pl
from jax.experimental.pallas import tpu as pltpu
from jax.experimental.pallas import tpu_sc as plsc

DIM = 64
SEQ = 50
JC = 2

_info = plsc.get_sparse_core_info()
_NC, _NS, _NL = _info.num_cores, _info.num_subcores, _info.num_lanes
_NW = _NC * _NS


@functools.partial(jax.jit, static_argnames=("n_seq",))
def _gather_sc(idx_flat, w, *, n_seq):
    s_per_w = n_seq // _NW
    b_per_w = s_per_w * SEQ
    nchunk = SEQ // JC
    chunk_rows = JC * s_per_w
    mesh = plsc.VectorSubcoreMesh(core_axis_name="c", subcore_axis_name="s")

    @functools.partial(
        pl.kernel,
        mesh=mesh,
        out_type=jax.ShapeDtypeStruct((SEQ, DIM, n_seq), jnp.float32),
        scratch_types=[
            pltpu.VMEM((b_per_w,), jnp.int32),
            pltpu.VMEM((b_per_w,), jnp.int32),
            pltpu.VMEM((chunk_rows, DIM), jnp.float32),
            pltpu.VMEM((chunk_rows, DIM), jnp.float32),
            pltpu.VMEM((DIM, JC * s_per_w + 1), jnp.float32),
            pltpu.VMEM((DIM, JC * s_per_w + 1), jnp.float32),
            pltpu.SemaphoreType.DMA,
            pltpu.SemaphoreType.DMA,
            pltpu.SemaphoreType.DMA,
            pltpu.SemaphoreType.DMA,
        ],
        compiler_params=pltpu.CompilerParams(
            use_tc_tiling_on_sc=False, needs_layout_passes=False
        ),
    )
    def k(idx_hbm, table_hbm, out_hbm, idx_v, idx_t, rows0, rows1, t0, t1,
          g0, g1, o0, o1):
        wid = lax.axis_index("s") * _NC + lax.axis_index("c")
        base = wid * b_per_w
        s0 = wid * s_per_w
        pltpu.sync_copy(idx_hbm.at[pl.ds(base, b_per_w)], idx_v)

        lanes = lax.iota(jnp.int32, _NL)
        nsb = s_per_w // _NL

        lane_seq = lanes * SEQ

        @plsc.parallel_loop(0, SEQ, unroll=2)
        def _build_idx(j):
            for sb in range(nsb):
                v = plsc.load_gather(idx_v, [lane_seq + (sb * _NL * SEQ + j)])
                plsc.store_scatter(
                    idx_t, [j * s_per_w + sb * _NL + lanes], v
                )

        bufs = (rows0, rows1)
        tbufs = (t0, t1)
        gsems = (g0, g1)
        osems = (o0, o1)

        def start_gather(c, buf, sem):
            return pltpu.async_copy(
                table_hbm.at[idx_t.at[pl.ds(c * chunk_rows, chunk_rows)]],
                buf, sem,
            )

        def transpose(buf, tbuf):
            dvecs = [d16 * _NL + lanes for d16 in range(DIM // _NL)]

            @plsc.parallel_loop(0, chunk_rows, unroll=8)
            def _tr(r):
                rsplat = jnp.full((_NL,), 0, jnp.int32) + r
                for d16, dvec in enumerate(dvecs):
                    v = buf[r, pl.ds(d16 * _NL, _NL)]
                    plsc.store_scatter(tbuf, [dvec, rsplat], v)

        def start_outs(c, tbuf, sem):
            copies = []
            for jl in range(JC):
                j = c * JC + jl
                copies.append(
                    pltpu.async_copy(
                        tbuf.at[:, pl.ds(jl * s_per_w, s_per_w)],
                        out_hbm.at[j, :, pl.ds(s0, s_per_w)],
                        sem,
                    )
                )
            return copies

        gathers = [None, None]
        outs = [None, None]
        gathers[0] = start_gather(0, bufs[0], gsems[0])
        for c in range(nchunk):
            b = c & 1
            gathers[b].wait()
            if c + 1 < nchunk:
                gathers[1 - b] = start_gather(c + 1, bufs[1 - b], gsems[1 - b])
            if outs[b] is not None:
                for o in outs[b]:
                    o.wait()
            transpose(bufs[b], tbufs[b])
            outs[b] = start_outs(c, tbufs[b], osems[b])
        for grp in outs:
            if grp is not None:
                for o in grp:
                    o.wait()

    return k(idx_flat, w)


def kernel(token_ids, w):
    n_seq = token_ids.shape[0]
    idx_flat = token_ids.reshape(-1).astype(jnp.int32)
    out_t = _gather_sc(idx_flat, w, n_seq=n_seq)
    return jnp.transpose(out_t, (2, 0, 1))
